# baseline (speedup 1.0000x reference)
import jax
import jax.numpy as jnp
from jax import lax
from jax.experimental import pallas as pl
from jax.experimental.pallas import tpu as pltpu
from jax.experimental.pallas import tpu_sc as plsc

N = 10000
NP = 10112
D = 128
E = 320000
CH = 64
NGW = 28
NDW = 4
NCHW = 180
EP = CH * NGW * NCHW
NCHUNK = EP // CH
NCHD = NCHUNK // NDW
ROWS_PER_TILE = NP // 16
NBUF = 5
NIBUF = 10
PROWS = NP // D


def _prep_body(feat_ref, mask_ref, w_ref, a_ref, q_ref, p_ref):
    f = feat_ref[...] * mask_ref[...][None, :]
    h = lax.dot_general(f, w_ref[...], (((1,), (1,)), ((), ())),
                        preferred_element_type=jnp.float32)
    t0 = lax.dot_general(h, a_ref[...], (((1,), (0,)), ((), ())),
                         preferred_element_type=jnp.float32)
    t = jnp.where(t0 > 0, t0, 0.01 * t0)
    m = jnp.max(t)
    p = jnp.exp(t - m)
    q_ref[...] = h * p
    p_ref[...] = p


_prep = pl.pallas_call(
    _prep_body,
    out_shape=[
        jax.ShapeDtypeStruct((NP, D), jnp.float32),
        jax.ShapeDtypeStruct((NP, 1), jnp.float32),
    ],
)


def _sc_body(idx_hbm, qtab_hbm, p_hbm, zeros_hbm,
             part_hbm, denp_hbm,
             i0, i1, i2, i3, i4, i5, i6, i7, i8, i9,
             r0, r1, r2, r3, r4, acc_sh,
             g0, g1, g2, g3, g4, s0, s1, s2, s3, s4,
             q0, q1, q2, q3, q4, q5, q6, q7, q8, q9):
    c = lax.axis_index("c")
    s = lax.axis_index("s")

    ibuf = (i0, i1, i2, i3, i4, i5, i6, i7, i8, i9)
    rows = (r0, r1, r2, r3, r4)
    gsem = (g0, g1, g2, g3, g4)
    ssem = (s0, s1, s2, s3, s4)
    isem = (q0, q1, q2, q3, q4, q5, q6, q7, q8, q9)

    pltpu.sync_copy(zeros_hbm.at[pl.ds(s * ROWS_PER_TILE, ROWS_PER_TILE)],
                    acc_sh.at[pl.ds(s * ROWS_PER_TILE, ROWS_PER_TILE)])
    plsc.subcore_barrier()

    @pl.when(s < 14)
    def _gather_role():
        base = (c * 14 + s) * NCHW

        def slot(ch, b, wait_sc, do_gather, do_prefetch):
            r, q = b % NBUF, b % NIBUF
            if wait_sc:
                pltpu.make_async_copy(
                    rows[(b - 2) % NBUF],
                    acc_sh.at[ibuf[(b - 2) % NIBUF].at[1]],
                    ssem[(b - 2) % NBUF]).wait()
            if do_gather:
                pltpu.make_async_copy(
                    idx_hbm.at[base + ch + 3], ibuf[(b + 3) % NIBUF],
                    isem[(b + 3) % NIBUF]).wait()
                pltpu.async_copy(qtab_hbm.at[ibuf[(b + 3) % NIBUF].at[0]],
                                 rows[(b + 3) % NBUF], gsem[(b + 3) % NBUF])
            pltpu.make_async_copy(qtab_hbm.at[ibuf[q].at[0]], rows[r],
                                  gsem[r]).wait()
            pltpu.async_copy(rows[r], acc_sh.at[ibuf[q].at[1]], ssem[r],
                             add=True)
            if do_prefetch:
                pltpu.async_copy(idx_hbm.at[base + ch + 5],
                                 ibuf[(b + 5) % NIBUF], isem[(b + 5) % NIBUF])

        for k in range(5):
            pltpu.async_copy(idx_hbm.at[base + k], ibuf[k], isem[k])
        for k in range(3):
            pltpu.make_async_copy(idx_hbm.at[base + k], ibuf[k],
                                  isem[k]).wait()
            pltpu.async_copy(qtab_hbm.at[ibuf[k].at[0]], rows[k], gsem[k])

        for b in range(10):
            slot(b, b, wait_sc=(b >= 2), do_gather=True, do_prefetch=True)

        def body(i, carry):
            ch = i * 10
            for b in range(10):
                slot(ch + b, b, wait_sc=True, do_gather=True,
                     do_prefetch=True)
            return carry

        lax.fori_loop(1, NCHW // 10 - 1, body, 0)

        last = NCHW - 10
        for b in range(10):
            slot(last + b, b, wait_sc=True,
                 do_gather=(b < 7), do_prefetch=(b < 5))

        for t in (NCHW - 2, NCHW - 1):
            pltpu.make_async_copy(rows[t % NBUF],
                                  acc_sh.at[ibuf[t % NIBUF].at[1]],
                                  ssem[t % NBUF]).wait()

    @pl.when(s >= 140)
    def _den_role():
        kbase = (c * 2 + (s - 14)) * NCHD

        pltpu.sync_copy(p_hbm, rows[0].at[pl.ds(0, PROWS)])

        def zbody(i, carry):
            for g in range(8):
                rows[1][i, pl.ds(g * 16, 16)] = jnp.zeros((16,), jnp.float32)
            return carry

        lax.fori_loop(0, PROWS, zbody, 0)

        def den_slot(ch, b, do_prefetch):
            q = b % NIBUF
            pltpu.make_async_copy(idx_hbm.at[kbase + ch], ibuf[q],
                                  isem[q]).wait()
            for g in range(CH // 16):
                si = ibuf[q][0, pl.ds(g * 16, 16)]
                di = ibuf[q][1, pl.ds(g * 16, 16)]
                pv = plsc.load_gather(
                    rows[0], [lax.shift_right_logical(si, 7), si & 127])
                plsc.addupdate_scatter(
                    rows[1], [lax.shift_right_logical(di, 7), di & 127], pv)
            if do_prefetch:
                pltpu.async_copy(idx_hbm.at[kbase + ch + 4],
                                 ibuf[(b + 4) % NIBUF], isem[(b + 4) % NIBUF])

        for k in range(4):
            pltpu.async_copy(idx_hbm.at[kbase + k], ibuf[k], isem[k])

        for b in range(8):
            den_slot(b, b, do_prefetch=True)

        def dbody(i, carry):
            ch = i * 8
            for b in range(8):
                den_slot(ch + b, b, do_prefetch=True)
            return carry

        lax.fori_loop(1, NCHD // 8 - 1, dbody, 0)

        last = NCHD - 8
        for b in range(8):
            den_slot(last + b, b, do_prefetch=(b < 4))

    plsc.subcore_barrier()

    pltpu.sync_copy(acc_sh.at[pl.ds(s * ROWS_PER_TILE, ROWS_PER_TILE)],
                    part_hbm.at[c, pl.ds(s * ROWS_PER_TILE, ROWS_PER_TILE)])

    @pl.when(s >= 14)
    def _den_wb():
        pltpu.sync_copy(rows[1].at[pl.ds(0, PROWS)],
                        denp_hbm.at[c * 2 + (s - 14)])


_sc_aggregate = pl.kernel(
    _sc_body,
    out_type=[
        jax.ShapeDtypeStruct((2, NP, D), jnp.float32),
        jax.ShapeDtypeStruct((NDW, PROWS, D), jnp.float32),
    ],
    mesh=plsc.VectorSubcoreMesh(core_axis_name="c", subcore_axis_name="s"),
    compiler_params=pltpu.CompilerParams(needs_layout_passes=False),
    scratch_types=(
        [pltpu.VMEM((2, CH), jnp.int32) for _ in range(NIBUF)]
        + [pltpu.VMEM((CH, D), jnp.float32) for _ in range(NBUF)]
        + [pltpu.VMEM_SHARED((NP, D), jnp.float32)]
        + [pltpu.SemaphoreType.DMA for _ in range(NBUF + NBUF + NIBUF)]
    ),
)


def _combine_body(part_ref, denp_ref, out_ref):
    num = part_ref[0, :N, :] + part_ref[1, :N, :]
    ones = jnp.ones((NDW, 1), jnp.float32)
    den = lax.dot_general(denp_ref[...], ones, (((0,), (0,)), ((), ())),
                          preferred_element_type=jnp.float32)
    den = den[:N, :]
    out_ref[...] = jnp.maximum(jnp.where(den != 0, num / den, 0.0), 0.0)


_combine = pl.pallas_call(
    _combine_body,
    out_shape=jax.ShapeDtypeStruct((N, D), jnp.float32),
)


@jax.jit
def kernel(feat, edge_index, mask, W, attn_param):
    feat_p = jnp.pad(feat, ((0, NP - N), (0, 0)))
    qtab, p = _prep(feat_p, mask, W, attn_param)

    src = jnp.pad(edge_index[0].astype(jnp.int32), (0, EP - E),
                  constant_values=N).reshape(NCHUNK, 1, CH)
    dst = jnp.pad(edge_index[1].astype(jnp.int32), (0, EP - E),
                  constant_values=N).reshape(NCHUNK, 1, CH)
    idx = jnp.concatenate([src, dst], axis=1)
    zeros = jnp.zeros((NP, D), jnp.float32)

    part, denp = _sc_aggregate(idx, qtab, p.reshape(PROWS, D), zeros)
    return _combine(part, denp.reshape(NDW, NP))


# speedup vs baseline: 1.1242x; 1.1242x over previous
import jax
import jax.numpy as jnp
from jax import lax
from jax.experimental import pallas as pl
from jax.experimental.pallas import tpu as pltpu
from jax.experimental.pallas import tpu_sc as plsc

N = 10000
NP = 10112
D = 128
E = 320000
CH = 80
NGW = 28
NDW = 4
NCHW = 144
EP = CH * NGW * NCHW
NCHUNK = EP // CH
NCHD = NCHUNK // NDW
ROWS_PER_TILE = NP // 16
NBUF = 4
NIBUF = 8
PROWS = NP // D


def _prep_body(feat_ref, mask_ref, w_ref, a_ref, q_ref, p_ref):
    f = feat_ref[...] * mask_ref[...][None, :]
    h = lax.dot_general(f, w_ref[...], (((1,), (1,)), ((), ())),
                        preferred_element_type=jnp.float32)
    t0 = lax.dot_general(h, a_ref[...], (((1,), (0,)), ((), ())),
                         preferred_element_type=jnp.float32)
    t = jnp.where(t0 > 0, t0, 0.01 * t0)
    m = jnp.max(t)
    p = jnp.exp(t - m)
    q_ref[...] = h * p
    p_ref[...] = p


_prep = pl.pallas_call(
    _prep_body,
    out_shape=[
        jax.ShapeDtypeStruct((NP, D), jnp.float32),
        jax.ShapeDtypeStruct((NP, 1), jnp.float32),
    ],
)


def _sc_body(idx_hbm, qtab_hbm, p_hbm,
             part_hbm, denp_hbm,
             i0, i1, i2, i3, i4, i5, i6, i7, r0, r1, r2, r3, acc_sh,
             g0, g1, g2, g3, s0, s1, s2, s3,
             q0, q1, q2, q3, q4, q5, q6, q7):
    c = lax.axis_index("c")
    s = lax.axis_index("s")

    ibuf = (i0, i1, i2, i3, i4, i5, i6, i7)
    rows = (r0, r1, r2, r3)
    gsem = (g0, g1, g2, g3)
    ssem = (s0, s1, s2, s3)
    isem = (q0, q1, q2, q3, q4, q5, q6, q7)

    def zacc(i, carry):
        for g in range(8):
            rows[0][i, pl.ds(g * 16, 16)] = jnp.zeros((16,), jnp.float32)
        return carry

    lax.fori_loop(0, CH, zacc, 0)
    for j in range(ROWS_PER_TILE // CH):
        pltpu.sync_copy(rows[0],
                        acc_sh.at[pl.ds(s * ROWS_PER_TILE + j * CH, CH)])
    _rem = ROWS_PER_TILE % CH
    pltpu.sync_copy(
        rows[0].at[pl.ds(0, _rem)],
        acc_sh.at[pl.ds(s * ROWS_PER_TILE + (ROWS_PER_TILE // CH) * CH,
                        _rem)])
    plsc.subcore_barrier()

    @pl.when(s < 14)
    def _gather_role():
        base = (c * 14 + s) * NCHW

        def slot(ch, b, wait_sc, do_gather, do_prefetch):
            r, q = b % NBUF, b % NIBUF
            if wait_sc:
                pltpu.make_async_copy(
                    rows[(b - 2) % NBUF],
                    acc_sh.at[ibuf[(b - 2) % NIBUF].at[1]],
                    ssem[(b - 2) % NBUF]).wait()
            if do_gather:
                pltpu.make_async_copy(
                    idx_hbm.at[base + ch + 2], ibuf[(b + 2) % NIBUF],
                    isem[(b + 2) % NIBUF]).wait()
                pltpu.async_copy(qtab_hbm.at[ibuf[(b + 2) % NIBUF].at[0]],
                                 rows[(b + 2) % NBUF], gsem[(b + 2) % NBUF])
            pltpu.make_async_copy(qtab_hbm.at[ibuf[q].at[0]], rows[r],
                                  gsem[r]).wait()
            pltpu.async_copy(rows[r], acc_sh.at[ibuf[q].at[1]], ssem[r],
                             add=True)
            if do_prefetch:
                pltpu.async_copy(idx_hbm.at[base + ch + 4],
                                 ibuf[(b + 4) % NIBUF], isem[(b + 4) % NIBUF])

        for k in range(4):
            pltpu.async_copy(idx_hbm.at[base + k], ibuf[k], isem[k])
        for k in range(2):
            pltpu.make_async_copy(idx_hbm.at[base + k], ibuf[k],
                                  isem[k]).wait()
            pltpu.async_copy(qtab_hbm.at[ibuf[k].at[0]], rows[k], gsem[k])

        for b in range(8):
            slot(b, b, wait_sc=(b >= 2), do_gather=True, do_prefetch=True)

        def body(i, carry):
            ch = i * 8
            for b in range(8):
                slot(ch + b, b, wait_sc=True, do_gather=True,
                     do_prefetch=True)
            return carry

        lax.fori_loop(1, NCHW // 8 - 1, body, 0)

        last = NCHW - 8
        for b in range(8):
            slot(last + b, b, wait_sc=True,
                 do_gather=(b < 6), do_prefetch=(b < 4))

        for t in (NCHW - 2, NCHW - 1):
            pltpu.make_async_copy(rows[t % NBUF],
                                  acc_sh.at[ibuf[t % NIBUF].at[1]],
                                  ssem[t % NBUF]).wait()

    @pl.when(s >= 14)
    def _den_role():
        kbase = (c * 2 + (s - 14)) * NCHD

        pltpu.sync_copy(p_hbm, rows[0].at[pl.ds(0, PROWS)])

        def zbody(i, carry):
            for g in range(8):
                rows[1][i, pl.ds(g * 16, 16)] = jnp.zeros((16,), jnp.float32)
            return carry

        lax.fori_loop(0, PROWS, zbody, 0)

        def den_slot(ch, b, do_prefetch):
            q = b % NIBUF
            pltpu.make_async_copy(idx_hbm.at[kbase + ch], ibuf[q],
                                  isem[q]).wait()
            for g in range(CH // 16):
                si = ibuf[q][0, pl.ds(g * 16, 16)]
                di = ibuf[q][1, pl.ds(g * 16, 16)]
                pv = plsc.load_gather(
                    rows[0], [lax.shift_right_logical(si, 7), si & 127])
                plsc.addupdate_scatter(
                    rows[1], [lax.shift_right_logical(di, 7), di & 127], pv)
            if do_prefetch:
                pltpu.async_copy(idx_hbm.at[kbase + ch + 4],
                                 ibuf[(b + 4) % NIBUF], isem[(b + 4) % NIBUF])

        for k in range(4):
            pltpu.async_copy(idx_hbm.at[kbase + k], ibuf[k], isem[k])

        for b in range(8):
            den_slot(b, b, do_prefetch=True)

        def dbody(i, carry):
            ch = i * 8
            for b in range(8):
                den_slot(ch + b, b, do_prefetch=True)
            return carry

        lax.fori_loop(1, NCHD // 8 - 1, dbody, 0)

        last = NCHD - 8
        for b in range(8):
            den_slot(last + b, b, do_prefetch=(b < 4))

    plsc.subcore_barrier()

    pltpu.sync_copy(acc_sh.at[pl.ds(s * ROWS_PER_TILE, ROWS_PER_TILE)],
                    part_hbm.at[c, pl.ds(s * ROWS_PER_TILE, ROWS_PER_TILE)])

    @pl.when(s >= 14)
    def _den_wb():
        pltpu.sync_copy(rows[1].at[pl.ds(0, PROWS)],
                        denp_hbm.at[c * 2 + (s - 14)])


_sc_aggregate = pl.kernel(
    _sc_body,
    out_type=[
        jax.ShapeDtypeStruct((2, NP, D), jnp.float32),
        jax.ShapeDtypeStruct((NDW, PROWS, D), jnp.float32),
    ],
    mesh=plsc.VectorSubcoreMesh(core_axis_name="c", subcore_axis_name="s"),
    compiler_params=pltpu.CompilerParams(needs_layout_passes=False),
    scratch_types=(
        [pltpu.VMEM((2, CH), jnp.int32) for _ in range(NIBUF)]
        + [pltpu.VMEM((CH, D), jnp.float32) for _ in range(NBUF)]
        + [pltpu.VMEM_SHARED((NP, D), jnp.float32)]
        + [pltpu.SemaphoreType.DMA for _ in range(NBUF + NBUF + NIBUF)]
    ),
)


def _combine_body(part_ref, denp_ref, out_ref):
    num = part_ref[0, :N, :] + part_ref[1, :N, :]
    ones = jnp.ones((NDW, 1), jnp.float32)
    den = lax.dot_general(denp_ref[...], ones, (((0,), (0,)), ((), ())),
                          preferred_element_type=jnp.float32)
    den = den[:N, :]
    out_ref[...] = jnp.maximum(jnp.where(den != 0, num / den, 0.0), 0.0)


_combine = pl.pallas_call(
    _combine_body,
    out_shape=jax.ShapeDtypeStruct((N, D), jnp.float32),
)


@jax.jit
def kernel(feat, edge_index, mask, W, attn_param):
    feat_p = jnp.pad(feat, ((0, NP - N), (0, 0)))
    qtab, p = _prep(feat_p, mask, W, attn_param)

    src = jnp.pad(edge_index[0].astype(jnp.int32), (0, EP - E),
                  constant_values=N).reshape(NCHUNK, 1, CH)
    dst = jnp.pad(edge_index[1].astype(jnp.int32), (0, EP - E),
                  constant_values=N).reshape(NCHUNK, 1, CH)
    idx = jnp.concatenate([src, dst], axis=1)
    part, denp = _sc_aggregate(idx, qtab, p.reshape(PROWS, D))
    return _combine(part, denp.reshape(NDW, NP))
